# baseline (speedup 1.0000x reference)
import functools

import jax
import jax.numpy as jnp
from jax import lax
from jax.experimental import pallas as pl
from jax.experimental.pallas import tpu as pltpu
from jax.experimental.pallas import tpu_sc as plsc

N = 10000
E = 320000
T = 4
F = 128
STEPS = 2

NC, NS = 2, 16
NW = NC * NS
CH = 64
K = 160
EPT = K * CH
EPAD = NW * EPT
PAD = EPAD - E
REAL_LAST = EPT - PAD
ROWS = N + 112
RPT = ROWS // NS


def _sc_make_idx_body(et_hbm, ei_hbm, idx_hbm, dstp_hbm,
                      et_v, idx_v, dst_v):
    wid = lax.axis_index("s") * NC + lax.axis_index("c")
    base = wid * EPT

    def _fill(n_real):
        def body(i, carry):
            for u in range(4):
                sl = pl.ds(i * 64 + u * 16, 16)
                idx_v[sl] = idx_v[sl] + et_v[sl] * N
            return carry

        lax.fori_loop(0, n_real // 64, body, 0)

    @pl.when(wid < NW - 1)
    def _():
        pltpu.sync_copy(et_hbm.at[pl.ds(base, EPT)], et_v)
        pltpu.sync_copy(ei_hbm.at[0, pl.ds(base, EPT)], idx_v)
        pltpu.sync_copy(ei_hbm.at[1, pl.ds(base, EPT)], dst_v)
        _fill(EPT)

    @pl.when(wid == NW - 1)
    def _():
        pltpu.sync_copy(et_hbm.at[pl.ds(base, REAL_LAST)],
                        et_v.at[pl.ds(0, REAL_LAST)])
        pltpu.sync_copy(ei_hbm.at[0, pl.ds(base, REAL_LAST)],
                        idx_v.at[pl.ds(0, REAL_LAST)])
        pltpu.sync_copy(ei_hbm.at[1, pl.ds(base, REAL_LAST)],
                        dst_v.at[pl.ds(0, REAL_LAST)])
        _fill(REAL_LAST)

        def pad_body(i, carry):
            sl = pl.ds(REAL_LAST + i * 16, 16)
            pidx = lax.iota(jnp.int32, 16) + i * 16
            idx_v[sl] = pidx
            dst_v[sl] = N + lax.rem(pidx, ROWS - N)
            return carry

        lax.fori_loop(0, PAD // 16, pad_body, 0)

    pltpu.sync_copy(idx_v, idx_hbm.at[pl.ds(base, EPT)])
    pltpu.sync_copy(dst_v, dstp_hbm.at[pl.ds(base, EPT)])


def _sc_segsum_body(table_hbm, idx_hbm, dst_hbm, out_hbm,
                    idx_v, dst_v, rows_a, rows_b, acc_sh, sem):
    cid = lax.axis_index("c")
    sid = lax.axis_index("s")
    wid = sid * NC + cid
    base = wid * EPT
    r0 = sid * RPT

    zv = jnp.zeros((16,), jnp.float32)

    def zbody(i, carry):
        for u in range(8):
            rows_a[i, pl.ds(u * 16, 16)] = zv
        return carry

    lax.fori_loop(0, CH, zbody, 0)
    for q in range(RPT // CH):
        pltpu.sync_copy(rows_a, acc_sh.at[pl.ds(r0 + q * CH, CH)])
    _TAIL = RPT - (RPT // CH) * CH
    if _TAIL:
        pltpu.sync_copy(rows_a.at[pl.ds(0, _TAIL)],
                        acc_sh.at[pl.ds(r0 + (RPT // CH) * CH, _TAIL)])
    pltpu.sync_copy(idx_hbm.at[pl.ds(base, EPT)], idx_v)
    pltpu.sync_copy(dst_hbm.at[wid], dst_v)
    plsc.subcore_barrier()

    def _gather(j, buf):
        pltpu.async_copy(
            table_hbm.at[idx_v.at[pl.ds(j * CH, CH)]], buf, sem)

    def _wait(buf):
        pltpu.make_async_copy(
            table_hbm.at[idx_v.at[pl.ds(0, CH)]], buf, sem).wait()

    _gather(0, rows_a)
    _gather(1, rows_b)

    def body(g, carry):
        j = g * 2
        _wait(rows_a)
        pltpu.sync_copy(rows_a, acc_sh.at[dst_v.at[j]], add=True)

        @pl.when(j + 2 < K)
        def _():
            _gather(j + 2, rows_a)

        _wait(rows_b)
        pltpu.sync_copy(rows_b, acc_sh.at[dst_v.at[j + 1]], add=True)

        @pl.when(j + 3 < K)
        def _():
            _gather(j + 3, rows_b)

        return carry

    lax.fori_loop(0, K // 2, body, 0)
    plsc.subcore_barrier()
    pltpu.sync_copy(acc_sh.at[pl.ds(r0, RPT)], out_hbm.at[cid, pl.ds(r0, RPT)])


@functools.cache
def _sc_kernels():
    mesh = plsc.VectorSubcoreMesh(core_axis_name="c", subcore_axis_name="s",
                                  num_cores=NC, num_subcores=NS)
    make_idx = pl.kernel(
        _sc_make_idx_body,
        out_type=(jax.ShapeDtypeStruct((EPAD,), jnp.int32),
                  jax.ShapeDtypeStruct((EPAD,), jnp.int32)),
        mesh=mesh,
        scratch_types=[
            pltpu.VMEM((EPT,), jnp.int32),
            pltpu.VMEM((EPT,), jnp.int32),
            pltpu.VMEM((EPT,), jnp.int32),
        ],
    )
    segsum = pl.kernel(
        _sc_segsum_body,
        out_type=jax.ShapeDtypeStruct((NC, ROWS, F), jnp.float32),
        mesh=mesh,
        scratch_types=[
            pltpu.VMEM((EPT,), jnp.int32),
            pltpu.VMEM((K, CH), jnp.int32),
            pltpu.VMEM((CH, F), jnp.float32),
            pltpu.VMEM((CH, F), jnp.float32),
            pltpu.VMEM_SHARED((ROWS, F), jnp.float32),
            pltpu.SemaphoreType.DMA,
        ],
    )
    return make_idx, segsum


_BN = 2000
_NB = N // _BN


def _proj_body(h_ref, w_ref, out_ref):
    out_ref[...] = jnp.dot(h_ref[...], w_ref[0],
                           preferred_element_type=jnp.float32)


_tc_proj = pl.pallas_call(
    _proj_body,
    grid=(T, _NB),
    in_specs=[
        pl.BlockSpec((_BN, F), lambda t, b: (b, 0)),
        pl.BlockSpec((1, F, F), lambda t, b: (t, 0, 0)),
    ],
    out_specs=pl.BlockSpec((_BN, F), lambda t, b: (t * _NB + b, 0)),
    out_shape=jax.ShapeDtypeStruct((T * N, F), jnp.float32),
)


def _gru_core(p0, p1, h, wih_ref, whh_ref, bih_ref, bhh_ref):
    a = p0 + p1
    gi = jnp.dot(a, wih_ref[...], preferred_element_type=jnp.float32) \
        + bih_ref[...]
    gh = jnp.dot(h, whh_ref[...], preferred_element_type=jnp.float32) \
        + bhh_ref[...]
    r = jax.nn.sigmoid(gi[:, 0:F] + gh[:, 0:F])
    z = jax.nn.sigmoid(gi[:, F:2 * F] + gh[:, F:2 * F])
    n = jnp.tanh(gi[:, 2 * F:3 * F] + r * gh[:, 2 * F:3 * F])
    return (1.0 - z) * n + z * h


def _gru_body(parts_ref0, parts_ref1, h_ref, wih_ref, whh_ref, bih_ref,
              bhh_ref, out_ref):
    out_ref[...] = _gru_core(parts_ref0[0], parts_ref1[0], h_ref[...],
                             wih_ref, whh_ref, bih_ref, bhh_ref)


def _gru_proj_body(parts_ref0, parts_ref1, h_ref, wih_ref, whh_ref, bih_ref,
                   bhh_ref, w_ref, out_ref, table_ref):
    h_new = _gru_core(parts_ref0[0], parts_ref1[0], h_ref[...],
                      wih_ref, whh_ref, bih_ref, bhh_ref)
    out_ref[...] = h_new
    for t in range(T):
        table_ref[t] = jnp.dot(h_new, w_ref[t],
                               preferred_element_type=jnp.float32)


_GRU_IN_SPECS = [
    pl.BlockSpec((1, _BN, F), lambda b: (0, b, 0)),
    pl.BlockSpec((1, _BN, F), lambda b: (1, b, 0)),
    pl.BlockSpec((_BN, F), lambda b: (b, 0)),
    pl.BlockSpec((F, 3 * F), lambda b: (0, 0)),
    pl.BlockSpec((F, 3 * F), lambda b: (0, 0)),
    pl.BlockSpec((1, 3 * F), lambda b: (0, 0)),
    pl.BlockSpec((1, 3 * F), lambda b: (0, 0)),
]

_tc_gru = pl.pallas_call(
    _gru_body,
    grid=(_NB,),
    in_specs=_GRU_IN_SPECS,
    out_specs=pl.BlockSpec((_BN, F), lambda b: (b, 0)),
    out_shape=jax.ShapeDtypeStruct((N, F), jnp.float32),
)

_tc_gru_proj = pl.pallas_call(
    _gru_proj_body,
    grid=(_NB,),
    in_specs=_GRU_IN_SPECS + [pl.BlockSpec((T, F, F), lambda b: (0, 0, 0))],
    out_specs=[
        pl.BlockSpec((_BN, F), lambda b: (b, 0)),
        pl.BlockSpec((T, _BN, F), lambda b: (0, b, 0)),
    ],
    out_shape=[
        jax.ShapeDtypeStruct((N, F), jnp.float32),
        jax.ShapeDtypeStruct((T, N, F), jnp.float32),
    ],
)


def kernel(feat, etypes, edge_index, weight, w_ih, w_hh, b_ih, b_hh):
    h = feat
    W = weight.reshape(T, F, F)
    wih_t = w_ih.T
    whh_t = w_hh.T
    bih = b_ih.reshape(1, 3 * F)
    bhh = b_hh.reshape(1, 3 * F)

    sc_make_idx, sc_segsum = _sc_kernels()
    idx, dst_p = sc_make_idx(etypes, edge_index)
    dst_p = dst_p.reshape(NW, K, CH)

    table = _tc_proj(h, W)
    parts = sc_segsum(table, idx, dst_p)
    h, table = _tc_gru_proj(parts, parts, h, wih_t, whh_t, bih, bhh, W)
    parts = sc_segsum(table.reshape(T * N, F), idx, dst_p)
    h = _tc_gru(parts, parts, h, wih_t, whh_t, bih, bhh)
    return h


# speedup vs baseline: 1.0369x; 1.0369x over previous
import functools

import jax
import jax.numpy as jnp
from jax import lax
from jax.experimental import pallas as pl
from jax.experimental.pallas import tpu as pltpu
from jax.experimental.pallas import tpu_sc as plsc

N = 10000
E = 320000
T = 4
F = 128
STEPS = 2

NC, NS = 2, 16
NW = NC * NS
CH = 64
K = 160
EPT = K * CH
EPAD = NW * EPT
PAD = EPAD - E
REAL_LAST = EPT - PAD
ROWS = N + 112
RPT = ROWS // NS


def _sc_make_idx_body(et_hbm, ei_hbm, pad_hbm, idx_hbm, dstp_hbm,
                      et_v, idx_v, dst_v, sem):
    wid = lax.axis_index("s") * NC + lax.axis_index("c")
    base = wid * EPT

    def _fill(n_real):
        def body(i, carry):
            for u in range(4):
                sl = pl.ds(i * 64 + u * 16, 16)
                idx_v[sl] = idx_v[sl] + et_v[sl] * N
            return carry

        lax.fori_loop(0, n_real // 64, body, 0)

    @pl.when(wid < NW - 1)
    def _():
        pltpu.async_copy(et_hbm.at[pl.ds(base, EPT)], et_v, sem)
        pltpu.async_copy(ei_hbm.at[0, pl.ds(base, EPT)], idx_v, sem)
        pltpu.async_copy(ei_hbm.at[1, pl.ds(base, EPT)], dst_v, sem)
        pltpu.make_async_copy(et_hbm.at[pl.ds(base, EPT)], et_v, sem).wait()
        pltpu.make_async_copy(et_hbm.at[pl.ds(base, EPT)], idx_v, sem).wait()
        pltpu.make_async_copy(et_hbm.at[pl.ds(base, EPT)], dst_v, sem).wait()
        _fill(EPT)

    @pl.when(wid == NW - 1)
    def _():
        pltpu.async_copy(et_hbm.at[pl.ds(base, REAL_LAST)],
                         et_v.at[pl.ds(0, REAL_LAST)], sem)
        pltpu.async_copy(ei_hbm.at[0, pl.ds(base, REAL_LAST)],
                         idx_v.at[pl.ds(0, REAL_LAST)], sem)
        pltpu.async_copy(ei_hbm.at[1, pl.ds(base, REAL_LAST)],
                         dst_v.at[pl.ds(0, REAL_LAST)], sem)
        pltpu.async_copy(pad_hbm.at[0], idx_v.at[pl.ds(REAL_LAST, PAD)], sem)
        pltpu.async_copy(pad_hbm.at[1], dst_v.at[pl.ds(REAL_LAST, PAD)], sem)
        for _ in range(3):
            pltpu.make_async_copy(
                et_hbm.at[pl.ds(base, REAL_LAST)], et_v.at[pl.ds(0, REAL_LAST)],
                sem).wait()
        for _ in range(2):
            pltpu.make_async_copy(
                pad_hbm.at[0], idx_v.at[pl.ds(REAL_LAST, PAD)], sem).wait()
        _fill(REAL_LAST)

    pltpu.sync_copy(idx_v, idx_hbm.at[pl.ds(base, EPT)])
    pltpu.sync_copy(dst_v, dstp_hbm.at[pl.ds(base, EPT)])


def _sc_segsum_body(table_hbm, idx_hbm, dst_hbm, out_hbm,
                    idx_v, dst_v, rows_a, rows_b, acc_sh, sem):
    cid = lax.axis_index("c")
    sid = lax.axis_index("s")
    wid = sid * NC + cid
    base = wid * EPT
    r0 = sid * RPT

    zv = jnp.zeros((16,), jnp.float32)

    def zbody(i, carry):
        for u in range(8):
            rows_a[i, pl.ds(u * 16, 16)] = zv
        return carry

    lax.fori_loop(0, CH, zbody, 0)
    for q in range(RPT // CH):
        pltpu.sync_copy(rows_a, acc_sh.at[pl.ds(r0 + q * CH, CH)])
    _TAIL = RPT - (RPT // CH) * CH
    if _TAIL:
        pltpu.sync_copy(rows_a.at[pl.ds(0, _TAIL)],
                        acc_sh.at[pl.ds(r0 + (RPT // CH) * CH, _TAIL)])
    pltpu.sync_copy(idx_hbm.at[pl.ds(base, EPT)], idx_v)
    pltpu.sync_copy(dst_hbm.at[wid], dst_v)
    plsc.subcore_barrier()

    def _gather(j, buf):
        pltpu.async_copy(
            table_hbm.at[idx_v.at[pl.ds(j * CH, CH)]], buf, sem)

    def _wait(buf):
        pltpu.make_async_copy(
            table_hbm.at[idx_v.at[pl.ds(0, CH)]], buf, sem).wait()

    _gather(0, rows_a)
    _gather(1, rows_b)

    def body(g, carry):
        j = g * 2
        _wait(rows_a)
        pltpu.sync_copy(rows_a, acc_sh.at[dst_v.at[j]], add=True)

        @pl.when(j + 2 < K)
        def _():
            _gather(j + 2, rows_a)

        _wait(rows_b)
        pltpu.sync_copy(rows_b, acc_sh.at[dst_v.at[j + 1]], add=True)

        @pl.when(j + 3 < K)
        def _():
            _gather(j + 3, rows_b)

        return carry

    lax.fori_loop(0, K // 2, body, 0)
    plsc.subcore_barrier()
    pltpu.sync_copy(acc_sh.at[pl.ds(r0, RPT)], out_hbm.at[cid, pl.ds(r0, RPT)])


@functools.cache
def _sc_kernels():
    mesh = plsc.VectorSubcoreMesh(core_axis_name="c", subcore_axis_name="s",
                                  num_cores=NC, num_subcores=NS)
    make_idx = pl.kernel(
        _sc_make_idx_body,
        out_type=(jax.ShapeDtypeStruct((EPAD,), jnp.int32),
                  jax.ShapeDtypeStruct((EPAD,), jnp.int32)),
        mesh=mesh,
        scratch_types=[
            pltpu.VMEM((EPT,), jnp.int32),
            pltpu.VMEM((EPT,), jnp.int32),
            pltpu.VMEM((EPT,), jnp.int32),
            pltpu.SemaphoreType.DMA,
        ],
    )
    segsum = pl.kernel(
        _sc_segsum_body,
        out_type=jax.ShapeDtypeStruct((NC, ROWS, F), jnp.float32),
        mesh=mesh,
        scratch_types=[
            pltpu.VMEM((EPT,), jnp.int32),
            pltpu.VMEM((K, CH), jnp.int32),
            pltpu.VMEM((CH, F), jnp.float32),
            pltpu.VMEM((CH, F), jnp.float32),
            pltpu.VMEM_SHARED((ROWS, F), jnp.float32),
            pltpu.SemaphoreType.DMA,
        ],
    )
    return make_idx, segsum


_BN = 2000
_NB = N // _BN


def _proj_body(h_ref, w_ref, out_ref):
    h = h_ref[...]
    for t in range(T):
        out_ref[t] = jnp.dot(h, w_ref[t], preferred_element_type=jnp.float32)


_tc_proj = pl.pallas_call(
    _proj_body,
    grid=(_NB,),
    in_specs=[
        pl.BlockSpec((_BN, F), lambda b: (b, 0)),
        pl.BlockSpec((T, F, F), lambda b: (0, 0, 0)),
    ],
    out_specs=pl.BlockSpec((T, _BN, F), lambda b: (0, b, 0)),
    out_shape=jax.ShapeDtypeStruct((T, N, F), jnp.float32),
)


def _gru_core(p0, p1, h, wih_ref, whh_ref, bih_ref, bhh_ref):
    a = p0 + p1
    gi = jnp.dot(a, wih_ref[...], preferred_element_type=jnp.float32) \
        + bih_ref[...]
    gh = jnp.dot(h, whh_ref[...], preferred_element_type=jnp.float32) \
        + bhh_ref[...]
    r = jax.nn.sigmoid(gi[:, 0:F] + gh[:, 0:F])
    z = jax.nn.sigmoid(gi[:, F:2 * F] + gh[:, F:2 * F])
    n = jnp.tanh(gi[:, 2 * F:3 * F] + r * gh[:, 2 * F:3 * F])
    return (1.0 - z) * n + z * h


def _gru_body(parts_ref0, parts_ref1, h_ref, wih_ref, whh_ref, bih_ref,
              bhh_ref, out_ref):
    out_ref[...] = _gru_core(parts_ref0[0], parts_ref1[0], h_ref[...],
                             wih_ref, whh_ref, bih_ref, bhh_ref)


def _gru_proj_body(parts_ref0, parts_ref1, h_ref, wih_ref, whh_ref, bih_ref,
                   bhh_ref, w_ref, out_ref, table_ref):
    h_new = _gru_core(parts_ref0[0], parts_ref1[0], h_ref[...],
                      wih_ref, whh_ref, bih_ref, bhh_ref)
    out_ref[...] = h_new
    for t in range(T):
        table_ref[t] = jnp.dot(h_new, w_ref[t],
                               preferred_element_type=jnp.float32)


_GRU_IN_SPECS = [
    pl.BlockSpec((1, _BN, F), lambda b: (0, b, 0)),
    pl.BlockSpec((1, _BN, F), lambda b: (1, b, 0)),
    pl.BlockSpec((_BN, F), lambda b: (b, 0)),
    pl.BlockSpec((F, 3 * F), lambda b: (0, 0)),
    pl.BlockSpec((F, 3 * F), lambda b: (0, 0)),
    pl.BlockSpec((1, 3 * F), lambda b: (0, 0)),
    pl.BlockSpec((1, 3 * F), lambda b: (0, 0)),
]

_tc_gru = pl.pallas_call(
    _gru_body,
    grid=(_NB,),
    in_specs=_GRU_IN_SPECS,
    out_specs=pl.BlockSpec((_BN, F), lambda b: (b, 0)),
    out_shape=jax.ShapeDtypeStruct((N, F), jnp.float32),
)

_tc_gru_proj = pl.pallas_call(
    _gru_proj_body,
    grid=(_NB,),
    in_specs=_GRU_IN_SPECS + [pl.BlockSpec((T, F, F), lambda b: (0, 0, 0))],
    out_specs=[
        pl.BlockSpec((_BN, F), lambda b: (b, 0)),
        pl.BlockSpec((T, _BN, F), lambda b: (0, b, 0)),
    ],
    out_shape=[
        jax.ShapeDtypeStruct((N, F), jnp.float32),
        jax.ShapeDtypeStruct((T, N, F), jnp.float32),
    ],
)


def kernel(feat, etypes, edge_index, weight, w_ih, w_hh, b_ih, b_hh):
    h = feat
    W = weight.reshape(T, F, F)
    wih_t = w_ih.T
    whh_t = w_hh.T
    bih = b_ih.reshape(1, 3 * F)
    bhh = b_hh.reshape(1, 3 * F)

    parange = jnp.arange(PAD, dtype=jnp.int32)
    pad_tail = jnp.stack([parange, N + parange % (ROWS - N)])

    sc_make_idx, sc_segsum = _sc_kernels()
    idx, dst_p = sc_make_idx(etypes, edge_index, pad_tail)
    dst_p = dst_p.reshape(NW, K, CH)

    table = _tc_proj(h, W).reshape(T * N, F)
    parts = sc_segsum(table, idx, dst_p)
    h, table = _tc_gru_proj(parts, parts, h, wih_t, whh_t, bih, bhh, W)
    parts = sc_segsum(table.reshape(T * N, F), idx, dst_p)
    h = _tc_gru(parts, parts, h, wih_t, whh_t, bih, bhh)
    return h


# speedup vs baseline: 1.1070x; 1.0676x over previous
import functools

import jax
import jax.numpy as jnp
from jax import lax
from jax.experimental import pallas as pl
from jax.experimental.pallas import tpu as pltpu
from jax.experimental.pallas import tpu_sc as plsc

N = 10000
E = 320000
T = 4
F = 128
STEPS = 2

NC, NS = 2, 16
NW = NC * NS
CH = 80
K = 128
EPT = K * CH
EPAD = NW * EPT
PAD = EPAD - E
REAL_LAST = EPT - PAD
ROWS = N + 112
RPT = ROWS // NS


def _sc_make_idx_body(et_hbm, ei_hbm, pad_hbm, idx_hbm, dstp_hbm,
                      et_v, idx_v, dst_v, sem):
    wid = lax.axis_index("s") * NC + lax.axis_index("c")
    base = wid * EPT

    def _fill(n_real):
        def body(i, carry):
            for u in range(4):
                sl = pl.ds(i * 64 + u * 16, 16)
                idx_v[sl] = idx_v[sl] + et_v[sl] * N
            return carry

        lax.fori_loop(0, n_real // 64, body, 0)

    @pl.when(wid < NW - 1)
    def _():
        pltpu.async_copy(et_hbm.at[pl.ds(base, EPT)], et_v, sem)
        pltpu.async_copy(ei_hbm.at[0, pl.ds(base, EPT)], idx_v, sem)
        pltpu.async_copy(ei_hbm.at[1, pl.ds(base, EPT)], dst_v, sem)
        pltpu.make_async_copy(et_hbm.at[pl.ds(base, EPT)], et_v, sem).wait()
        pltpu.make_async_copy(et_hbm.at[pl.ds(base, EPT)], idx_v, sem).wait()
        pltpu.make_async_copy(et_hbm.at[pl.ds(base, EPT)], dst_v, sem).wait()
        _fill(EPT)

    @pl.when(wid == NW - 1)
    def _():
        pltpu.async_copy(et_hbm.at[pl.ds(base, REAL_LAST)],
                         et_v.at[pl.ds(0, REAL_LAST)], sem)
        pltpu.async_copy(ei_hbm.at[0, pl.ds(base, REAL_LAST)],
                         idx_v.at[pl.ds(0, REAL_LAST)], sem)
        pltpu.async_copy(ei_hbm.at[1, pl.ds(base, REAL_LAST)],
                         dst_v.at[pl.ds(0, REAL_LAST)], sem)
        pltpu.async_copy(pad_hbm.at[0], idx_v.at[pl.ds(REAL_LAST, PAD)], sem)
        pltpu.async_copy(pad_hbm.at[1], dst_v.at[pl.ds(REAL_LAST, PAD)], sem)
        for _ in range(3):
            pltpu.make_async_copy(
                et_hbm.at[pl.ds(base, REAL_LAST)], et_v.at[pl.ds(0, REAL_LAST)],
                sem).wait()
        for _ in range(2):
            pltpu.make_async_copy(
                pad_hbm.at[0], idx_v.at[pl.ds(REAL_LAST, PAD)], sem).wait()
        _fill(REAL_LAST)

    pltpu.sync_copy(idx_v, idx_hbm.at[pl.ds(base, EPT)])
    pltpu.sync_copy(dst_v, dstp_hbm.at[pl.ds(base, EPT)])


def _sc_segsum_body(table_hbm, idx_hbm, dst_hbm, out_hbm,
                    idx_v, dst_v, rows_a, rows_b, acc_sh, sem):
    cid = lax.axis_index("c")
    sid = lax.axis_index("s")
    wid = sid * NC + cid
    base = wid * EPT
    r0 = sid * RPT

    zv = jnp.zeros((16,), jnp.float32)

    def zbody(i, carry):
        for u in range(8):
            rows_a[i, pl.ds(u * 16, 16)] = zv
        return carry

    lax.fori_loop(0, CH, zbody, 0)
    for q in range(RPT // CH):
        pltpu.sync_copy(rows_a, acc_sh.at[pl.ds(r0 + q * CH, CH)])
    _TAIL = RPT - (RPT // CH) * CH
    if _TAIL:
        pltpu.sync_copy(rows_a.at[pl.ds(0, _TAIL)],
                        acc_sh.at[pl.ds(r0 + (RPT // CH) * CH, _TAIL)])
    pltpu.sync_copy(idx_hbm.at[pl.ds(base, EPT)], idx_v)
    pltpu.sync_copy(dst_hbm.at[wid], dst_v)
    plsc.subcore_barrier()

    def _gather(j, buf):
        pltpu.async_copy(
            table_hbm.at[idx_v.at[pl.ds(j * CH, CH)]], buf, sem)

    def _wait(buf):
        pltpu.make_async_copy(
            table_hbm.at[idx_v.at[pl.ds(0, CH)]], buf, sem).wait()

    _gather(0, rows_a)
    _gather(1, rows_b)

    def stage(j, cur):
        _wait(cur)
        pltpu.sync_copy(cur, acc_sh.at[dst_v.at[j]], add=True)

        @pl.when(j + 2 < K)
        def _():
            _gather(j + 2, cur)

    def body(g, carry):
        j = g * 2
        stage(j, rows_a)
        stage(j + 1, rows_b)
        return carry

    lax.fori_loop(0, K // 2, body, 0)
    plsc.subcore_barrier()
    pltpu.sync_copy(acc_sh.at[pl.ds(r0, RPT)], out_hbm.at[cid, pl.ds(r0, RPT)])


@functools.cache
def _sc_kernels():
    mesh = plsc.VectorSubcoreMesh(core_axis_name="c", subcore_axis_name="s",
                                  num_cores=NC, num_subcores=NS)
    make_idx = pl.kernel(
        _sc_make_idx_body,
        out_type=(jax.ShapeDtypeStruct((EPAD,), jnp.int32),
                  jax.ShapeDtypeStruct((EPAD,), jnp.int32)),
        mesh=mesh,
        scratch_types=[
            pltpu.VMEM((EPT,), jnp.int32),
            pltpu.VMEM((EPT,), jnp.int32),
            pltpu.VMEM((EPT,), jnp.int32),
            pltpu.SemaphoreType.DMA,
        ],
    )
    segsum = pl.kernel(
        _sc_segsum_body,
        out_type=jax.ShapeDtypeStruct((NC, ROWS, F), jnp.float32),
        mesh=mesh,
        scratch_types=[
            pltpu.VMEM((EPT,), jnp.int32),
            pltpu.VMEM((K, CH), jnp.int32),
            pltpu.VMEM((CH, F), jnp.float32),
            pltpu.VMEM((CH, F), jnp.float32),
            pltpu.VMEM_SHARED((ROWS, F), jnp.float32),
            pltpu.SemaphoreType.DMA,
        ],
    )
    return make_idx, segsum


_BN = 2000
_NB = N // _BN


def _proj_body(h_ref, w_ref, out_ref):
    h = h_ref[...]
    for t in range(T):
        out_ref[t] = jnp.dot(h, w_ref[t], preferred_element_type=jnp.float32)


_tc_proj = pl.pallas_call(
    _proj_body,
    grid=(_NB,),
    in_specs=[
        pl.BlockSpec((_BN, F), lambda b: (b, 0)),
        pl.BlockSpec((T, F, F), lambda b: (0, 0, 0)),
    ],
    out_specs=pl.BlockSpec((T, _BN, F), lambda b: (0, b, 0)),
    out_shape=jax.ShapeDtypeStruct((T, N, F), jnp.float32),
)


def _gru_core(p0, p1, h, wih_ref, whh_ref, bih_ref, bhh_ref):
    a = p0 + p1
    gi = jnp.dot(a, wih_ref[...], preferred_element_type=jnp.float32) \
        + bih_ref[...]
    gh = jnp.dot(h, whh_ref[...], preferred_element_type=jnp.float32) \
        + bhh_ref[...]
    r = jax.nn.sigmoid(gi[:, 0:F] + gh[:, 0:F])
    z = jax.nn.sigmoid(gi[:, F:2 * F] + gh[:, F:2 * F])
    n = jnp.tanh(gi[:, 2 * F:3 * F] + r * gh[:, 2 * F:3 * F])
    return (1.0 - z) * n + z * h


def _gru_body(parts_ref0, parts_ref1, h_ref, wih_ref, whh_ref, bih_ref,
              bhh_ref, out_ref):
    out_ref[...] = _gru_core(parts_ref0[0], parts_ref1[0], h_ref[...],
                             wih_ref, whh_ref, bih_ref, bhh_ref)


def _gru_proj_body(parts_ref0, parts_ref1, h_ref, wih_ref, whh_ref, bih_ref,
                   bhh_ref, w_ref, out_ref, table_ref):
    h_new = _gru_core(parts_ref0[0], parts_ref1[0], h_ref[...],
                      wih_ref, whh_ref, bih_ref, bhh_ref)
    out_ref[...] = h_new
    for t in range(T):
        table_ref[t] = jnp.dot(h_new, w_ref[t],
                               preferred_element_type=jnp.float32)


_GRU_IN_SPECS = [
    pl.BlockSpec((1, _BN, F), lambda b: (0, b, 0)),
    pl.BlockSpec((1, _BN, F), lambda b: (1, b, 0)),
    pl.BlockSpec((_BN, F), lambda b: (b, 0)),
    pl.BlockSpec((F, 3 * F), lambda b: (0, 0)),
    pl.BlockSpec((F, 3 * F), lambda b: (0, 0)),
    pl.BlockSpec((1, 3 * F), lambda b: (0, 0)),
    pl.BlockSpec((1, 3 * F), lambda b: (0, 0)),
]

_tc_gru = pl.pallas_call(
    _gru_body,
    grid=(_NB,),
    in_specs=_GRU_IN_SPECS,
    out_specs=pl.BlockSpec((_BN, F), lambda b: (b, 0)),
    out_shape=jax.ShapeDtypeStruct((N, F), jnp.float32),
)

_tc_gru_proj = pl.pallas_call(
    _gru_proj_body,
    grid=(_NB,),
    in_specs=_GRU_IN_SPECS + [pl.BlockSpec((T, F, F), lambda b: (0, 0, 0))],
    out_specs=[
        pl.BlockSpec((_BN, F), lambda b: (b, 0)),
        pl.BlockSpec((T, _BN, F), lambda b: (0, b, 0)),
    ],
    out_shape=[
        jax.ShapeDtypeStruct((N, F), jnp.float32),
        jax.ShapeDtypeStruct((T, N, F), jnp.float32),
    ],
)


def kernel(feat, etypes, edge_index, weight, w_ih, w_hh, b_ih, b_hh):
    h = feat
    W = weight.reshape(T, F, F)
    wih_t = w_ih.T
    whh_t = w_hh.T
    bih = b_ih.reshape(1, 3 * F)
    bhh = b_hh.reshape(1, 3 * F)

    parange = jnp.arange(PAD, dtype=jnp.int32)
    pad_tail = jnp.stack([parange, N + parange % (ROWS - N)])

    sc_make_idx, sc_segsum = _sc_kernels()
    idx, dst_p = sc_make_idx(etypes, edge_index, pad_tail)
    dst_p = dst_p.reshape(NW, K, CH)

    table = _tc_proj(h, W).reshape(T * N, F)
    parts = sc_segsum(table, idx, dst_p)
    h, table = _tc_gru_proj(parts, parts, h, wih_t, whh_t, bih, bhh, W)
    parts = sc_segsum(table.reshape(T * N, F), idx, dst_p)
    h = _tc_gru(parts, parts, h, wih_t, whh_t, bih, bhh)
    return h
